# baseline (speedup 1.0000x reference)
import functools

import jax
import jax.numpy as jnp
from jax import lax
from jax.experimental import pallas as pl
from jax.experimental.pallas import tpu as pltpu
from jax.experimental.pallas import tpu_sc as plsc

_CHUNK = 64
_DEPTH = 2


@functools.lru_cache(maxsize=None)
def _build(B, V, D, idx_dtype):
    info = plsc.get_sparse_core_info()
    nw = info.num_cores * info.num_subcores
    b_per_w = B // nw
    nchunk = b_per_w // _CHUNK
    mesh = plsc.VectorSubcoreMesh(core_axis_name="c", subcore_axis_name="s")

    @functools.partial(
        pl.kernel,
        mesh=mesh,
        compiler_params=pltpu.CompilerParams(use_tc_tiling_on_sc=False),
        out_type=jax.ShapeDtypeStruct((B, D), jnp.float32),
        scratch_types=[
            pltpu.VMEM((nchunk, _CHUNK), jnp.int32),
            pltpu.VMEM((b_per_w, D), jnp.float32),
            [pltpu.SemaphoreType.DMA] * nchunk,
        ],
    )
    def gather_kernel(idx_hbm, table_hbm, out_hbm, idx_v, rows_v, sems):
        wid = lax.axis_index("s") * info.num_cores + lax.axis_index("c")
        base = wid * b_per_w

        def gather(j):
            return pltpu.async_copy(
                table_hbm.at[idx_v.at[j]],
                rows_v.at[pl.ds(j * _CHUNK, _CHUNK)],
                sems[j],
            )

        def store(j):
            return pltpu.async_copy(
                rows_v.at[pl.ds(j * _CHUNK, _CHUNK)],
                out_hbm.at[pl.ds(base + j * _CHUNK, _CHUNK)],
                sems[j],
            )

        pltpu.sync_copy(idx_hbm.at[wid], idx_v)
        gathers = [gather(j) for j in range(_DEPTH)]
        stores = []
        for j in range(nchunk):
            gathers[j].wait()
            stores.append(store(j))
            if j + _DEPTH < nchunk:
                gathers.append(gather(j + _DEPTH))
        for s in stores:
            s.wait()

    def run(indexes, entity_table):
        idx3 = indexes.astype(jnp.int32).reshape(nw, nchunk, _CHUNK)
        return gather_kernel(idx3, entity_table)

    return run


def kernel(indexes, entity_table):
    (B,) = indexes.shape
    V, D = entity_table.shape
    return _build(B, V, D, indexes.dtype.name)(indexes, entity_table)


# speedup vs baseline: 1.0115x; 1.0115x over previous
import functools

import jax
import jax.numpy as jnp
from jax import lax
from jax.experimental import pallas as pl
from jax.experimental.pallas import tpu as pltpu
from jax.experimental.pallas import tpu_sc as plsc

_CHUNK = 128


@functools.lru_cache(maxsize=None)
def _build(B, V, D, idx_dtype):
    info = plsc.get_sparse_core_info()
    nw = info.num_cores * info.num_subcores
    b_per_w = B // nw
    nchunk = b_per_w // _CHUNK
    mesh = plsc.VectorSubcoreMesh(core_axis_name="c", subcore_axis_name="s")

    @functools.partial(
        pl.kernel,
        mesh=mesh,
        out_type=jax.ShapeDtypeStruct((B, 2 * D), jnp.float32),
        scratch_types=[
            pltpu.VMEM((nchunk, _CHUNK), jnp.int32),
            pltpu.VMEM((nchunk, _CHUNK), jnp.int32),
            pltpu.VMEM((b_per_w, 2 * D), jnp.float32),
            pltpu.SemaphoreType.DMA,
        ],
    )
    def gather_kernel(idx_hbm, table_hbm, out_hbm, idx_v, qidx_v, rows_v, sem):
        wid = lax.axis_index("s") * info.num_cores + lax.axis_index("c")
        base = pl.multiple_of(wid * b_per_w, 8)
        pltpu.sync_copy(idx_hbm.at[wid], idx_v)
        for j in range(nchunk):
            for k in range(_CHUNK // 16):
                r = idx_v[j, pl.ds(k * 16, 16)]
                qidx_v[j, pl.ds(k * 16, 16)] = lax.shift_right_logical(r, 1)
        copies = [
            pltpu.async_copy(
                table_hbm.at[qidx_v.at[j]],
                rows_v.at[pl.ds(j * _CHUNK, _CHUNK)],
                sem,
            )
            for j in range(nchunk)
        ]
        for c in copies:
            c.wait()
        pltpu.sync_copy(rows_v, out_hbm.at[pl.ds(base, b_per_w)])

    def run(indexes, entity_table):
        idx3 = indexes.astype(jnp.int32).reshape(nw, nchunk, _CHUNK)
        table2 = entity_table.reshape(V // 2, 2 * D)
        out2 = gather_kernel(idx3, table2)
        return out2[:, :D]

    return run


def kernel(indexes, entity_table):
    (B,) = indexes.shape
    V, D = entity_table.shape
    return _build(B, V, D, indexes.dtype.name)(indexes, entity_table)


# speedup vs baseline: 1.5352x; 1.5177x over previous
import functools

import jax
import jax.numpy as jnp
from jax import lax
from jax.experimental import pallas as pl
from jax.experimental.pallas import tpu as pltpu
from jax.experimental.pallas import tpu_sc as plsc

_CHUNK = 128
_RBLK = 3200


def _tc_transpose(tT, V, D):
    nblk = pl.cdiv(V, _RBLK)

    def body(in_ref, out_ref):
        out_ref[:, 0:D] = in_ref[...].T

    return pl.pallas_call(
        body,
        grid=(nblk,),
        in_specs=[pl.BlockSpec((D, _RBLK), lambda i: (0, i))],
        out_specs=pl.BlockSpec((_RBLK, 2 * D), lambda i: (i, 0)),
        out_shape=jax.ShapeDtypeStruct((V, 2 * D), jnp.float32),
    )(tT)


@functools.lru_cache(maxsize=None)
def _build(B, V, D, idx_dtype):
    info = plsc.get_sparse_core_info()
    nw = info.num_cores * info.num_subcores
    b_per_w = B // nw
    nchunk = b_per_w // _CHUNK
    ngrp = b_per_w // 16
    mesh = plsc.VectorSubcoreMesh(core_axis_name="c", subcore_axis_name="s")

    @functools.partial(
        pl.kernel,
        mesh=mesh,
        compiler_params=pltpu.CompilerParams(use_tc_tiling_on_sc=False),
        out_type=jax.ShapeDtypeStruct((B // 2, 2 * D), jnp.float32),
        scratch_types=[
            pltpu.VMEM((nchunk, _CHUNK), jnp.int32),
            pltpu.VMEM((b_per_w, 2 * D), jnp.float32),
            pltpu.VMEM((b_per_w // 2, 2 * D), jnp.float32),
            pltpu.SemaphoreType.DMA,
        ],
    )
    def gather_kernel(idx_hbm, table_hbm, out_hbm, idx_v, rows_v, pack_v, sem):
        wid = lax.axis_index("s") * info.num_cores + lax.axis_index("c")
        pltpu.sync_copy(idx_hbm.at[wid], idx_v)
        copies = [
            pltpu.async_copy(
                table_hbm.at[idx_v.at[j]],
                rows_v.at[pl.ds(j * _CHUNK, _CHUNK)],
                sem,
            )
            for j in range(nchunk)
        ]
        for c in copies:
            c.wait()

        for i in range(b_per_w):
            for k in range(D // 16):
                pack_v[i // 2, pl.ds((i % 2) * D + k * 16, 16)] = rows_v[
                    i, pl.ds(k * 16, 16)
                ]
        obase = pl.multiple_of(wid * (b_per_w // 2), 8)
        pltpu.sync_copy(pack_v, out_hbm.at[pl.ds(obase, b_per_w // 2)])

    def run(indexes, entity_table):
        idx3 = indexes.astype(jnp.int32).reshape(nw, nchunk, _CHUNK)
        table2 = _tc_transpose(entity_table.T, V, D)
        out2 = gather_kernel(idx3, table2)
        return out2.reshape(B, D)

    return run


def kernel(indexes, entity_table):
    (B,) = indexes.shape
    V, D = entity_table.shape
    return _build(B, V, D, indexes.dtype.name)(indexes, entity_table)


# speedup vs baseline: 2.2029x; 1.4349x over previous
import functools

import jax
import jax.numpy as jnp
from jax import lax
from jax.experimental import pallas as pl
from jax.experimental.pallas import tpu as pltpu
from jax.experimental.pallas import tpu_sc as plsc

_CHUNK = 128
_RBLK = 12800


def _tc_transpose(tT, V, D):
    nblk = pl.cdiv(V, _RBLK)

    def body(in_ref, out_ref):
        out_ref[:, 0:D] = in_ref[...].T

    return pl.pallas_call(
        body,
        grid=(nblk,),
        in_specs=[pl.BlockSpec((D, _RBLK), lambda i: (0, i))],
        out_specs=pl.BlockSpec((_RBLK, 2 * D), lambda i: (i, 0)),
        out_shape=jax.ShapeDtypeStruct((V, 2 * D), jnp.float32),
    )(tT)


@functools.lru_cache(maxsize=None)
def _build(B, V, D, idx_dtype):
    info = plsc.get_sparse_core_info()
    nw = info.num_cores * info.num_subcores
    b_per_w = B // nw
    nchunk = b_per_w // _CHUNK
    ngrp = b_per_w // 16
    mesh = plsc.VectorSubcoreMesh(core_axis_name="c", subcore_axis_name="s")

    @functools.partial(
        pl.kernel,
        mesh=mesh,
        compiler_params=pltpu.CompilerParams(use_tc_tiling_on_sc=False),
        out_type=jax.ShapeDtypeStruct((B // 2, 2 * D), jnp.float32),
        scratch_types=[
            pltpu.VMEM((nchunk, _CHUNK), jnp.int32),
            pltpu.VMEM((b_per_w, 2 * D), jnp.float32),
            pltpu.VMEM((b_per_w // 2, 2 * D), jnp.float32),
            pltpu.SemaphoreType.DMA,
        ],
    )
    def gather_kernel(idx_hbm, table_hbm, out_hbm, idx_v, rows_v, pack_v, sem):
        wid = lax.axis_index("s") * info.num_cores + lax.axis_index("c")
        pltpu.sync_copy(idx_hbm.at[wid], idx_v)
        copies = [
            pltpu.async_copy(
                table_hbm.at[idx_v.at[j]],
                rows_v.at[pl.ds(j * _CHUNK, _CHUNK)],
                sem,
            )
            for j in range(nchunk)
        ]
        for c in copies:
            c.wait()

        for i in range(b_per_w):
            for k in range(D // 16):
                pack_v[i // 2, pl.ds((i % 2) * D + k * 16, 16)] = rows_v[
                    i, pl.ds(k * 16, 16)
                ]
        obase = pl.multiple_of(wid * (b_per_w // 2), 8)
        pltpu.sync_copy(pack_v, out_hbm.at[pl.ds(obase, b_per_w // 2)])

    def run(indexes, entity_table):
        idx3 = indexes.astype(jnp.int32).reshape(nw, nchunk, _CHUNK)
        table2 = _tc_transpose(entity_table.T, V, D)
        out2 = gather_kernel(idx3, table2)
        return out2.reshape(B, D)

    return run


def kernel(indexes, entity_table):
    (B,) = indexes.shape
    V, D = entity_table.shape
    return _build(B, V, D, indexes.dtype.name)(indexes, entity_table)
